# baseline (speedup 1.0000x reference)
import functools

import jax
import jax.numpy as jnp
from jax import lax
from jax.experimental import pallas as pl
from jax.experimental.pallas import tpu as pltpu
from jax.experimental.pallas import tpu_sc as plsc

BATCH = 16384
BIT = 64
L = 16
NC = 2
NS = 16
NW = NC * NS
NTRAIN = 100000
PACK_BLK = 12288
PACK_GRID = 9
BPW = BATCH // NS
UCHUNK = 256
NCHUNK = BPW // UCHUNK


def _pack_body(ht_ref, p0_ref, p1_ref):
    x = ht_ref[...]
    j = lax.broadcasted_iota(jnp.int32, (BIT, 1), 0)
    w = jnp.left_shift(jnp.int32(1), j % 32)
    contrib = jnp.where(x < 0.0, w, 0)
    p0_ref[...] = jnp.sum(contrib[:32, :], axis=0)
    p1_ref[...] = jnp.sum(contrib[32:, :], axis=0)


_pack = pl.pallas_call(
    _pack_body,
    grid=(PACK_GRID,),
    in_specs=[pl.BlockSpec((BIT, PACK_BLK), lambda i: (0, i))],
    out_specs=[
        pl.BlockSpec((PACK_BLK,), lambda i: (i,)),
        pl.BlockSpec((PACK_BLK,), lambda i: (i,)),
    ],
    out_shape=[
        jax.ShapeDtypeStruct((NTRAIN,), jnp.int32),
        jax.ShapeDtypeStruct((NTRAIN,), jnp.int32),
    ],
)

_mesh = plsc.VectorSubcoreMesh(core_axis_name="c", subcore_axis_name="s")


@functools.partial(
    pl.kernel,
    out_type=jax.ShapeDtypeStruct((NW, L), jnp.float32),
    mesh=_mesh,
    scratch_types=[
        pltpu.VMEM((NTRAIN,), jnp.int32),
        pltpu.VMEM((BPW,), jnp.int32),
        pltpu.VMEM((2, BIT // 2, UCHUNK), jnp.float32),
        pltpu.VMEM((L,), jnp.float32),
        pltpu.SemaphoreType.DMA,
        pltpu.SemaphoreType.DMA,
        pltpu.SemaphoreType.DMA,
    ],
    compiler_params=pltpu.CompilerParams(needs_layout_passes=False),
)
def _mse_bits(
    ut_hbm, ind_hbm, p0_hbm, p1_hbm, out_hbm, pg_v, ind_v, ut_v, acc_v,
    psem, isem, usem,
):
    g = lax.axis_index("c")
    s = lax.axis_index("s")
    wid = s * NC + g
    jbase = pl.multiple_of(g * (BIT // 2), BIT // 2)
    ibase = pl.multiple_of(s * BPW, BPW)

    pcopy0 = pltpu.make_async_copy(p0_hbm, pg_v, psem)
    pcopy1 = pltpu.make_async_copy(p1_hbm, pg_v, psem)

    @pl.when(g == 0)
    def _():
        pcopy0.start()

    @pl.when(g != 0)
    def _():
        pcopy1.start()

    icopy = pltpu.make_async_copy(ind_hbm.at[pl.ds(ibase, BPW)], ind_v, isem)
    icopy.start()

    def u_copy(c, buf):
        return pltpu.make_async_copy(
            ut_hbm.at[
                pl.ds(jbase, BIT // 2),
                pl.ds(pl.multiple_of(ibase + c * UCHUNK, UCHUNK), UCHUNK),
            ],
            ut_v.at[buf],
            usem,
        )

    u_copy(0, 0).start()
    u_copy(1, 1).start()
    icopy.wait()
    pcopy0.wait()

    zero = jnp.zeros((L,), jnp.float32)
    accs = (zero, zero, zero, zero)
    for c in range(NCHUNK):
        buf = c % 2
        u_copy(c, buf).wait()
        cbase = c * UCHUNK

        @plsc.parallel_loop(0, UCHUNK, L, carry=accs)
        def body(k, a, cbase=cbase, buf=buf):
            idx = ind_v[pl.ds(cbase + k, L)]
            p = plsc.load_gather(pg_v, [idx])
            a = list(a)
            for jj in range(BIT // 2):
                neg = jnp.left_shift(p, 31 - jj) < 0
                h = jnp.where(neg, jnp.float32(-1.0), jnp.float32(1.0))
                d = ut_v[buf, jj, pl.ds(k, L)] - h
                a[jj % 4] = a[jj % 4] + d * d
            return tuple(a)

        accs = body
        if c + 2 < NCHUNK:
            u_copy(c + 2, buf).start()
    acc_v[...] = (accs[0] + accs[1]) + (accs[2] + accs[3])
    pltpu.sync_copy(acc_v, out_hbm.at[wid])


def kernel(u, y, ind, H):
    del y
    p0, p1 = _pack(H.T)
    partials = _mse_bits(u.T, ind.astype(jnp.int32), p0, p1)
    return jnp.sum(partials) * (1.0 / (BATCH * BIT))


# speedup vs baseline: 1.0136x; 1.0136x over previous
import functools

import jax
import jax.numpy as jnp
from jax import lax
from jax.experimental import pallas as pl
from jax.experimental.pallas import tpu as pltpu
from jax.experimental.pallas import tpu_sc as plsc

BATCH = 16384
BIT = 64
L = 16
NC = 2
NS = 16
NW = NC * NS
NTRAIN = 100000
PACK_BLK = 12288
PACK_GRID = 9
BPW = BATCH // NS
UCHUNK = 256
NCHUNK = BPW // UCHUNK


def _pack_body(ht_ref, p0_ref, p1_ref):
    x = ht_ref[...]
    j = lax.broadcasted_iota(jnp.int32, (4, BIT), 1)
    q = lax.broadcasted_iota(jnp.int32, (4, BIT), 0)
    inq = (j >= q * 16) & (j < (q + 1) * 16)
    w = jnp.where(inq, jnp.left_shift(jnp.int32(1), j % 16), 0).astype(
        jnp.float32
    )
    dot = lax.dot_general(
        w, x, (((1,), (0,)), ((), ())), preferred_element_type=jnp.float32
    )
    p16 = ((jnp.float32(65535.0) - dot) * jnp.float32(0.5)).astype(jnp.int32)
    p0_ref[...] = p16[0, :] + jnp.left_shift(p16[1, :], 16)
    p1_ref[...] = p16[2, :] + jnp.left_shift(p16[3, :], 16)


_pack = pl.pallas_call(
    _pack_body,
    grid=(PACK_GRID,),
    in_specs=[pl.BlockSpec((BIT, PACK_BLK), lambda i: (0, i))],
    out_specs=[
        pl.BlockSpec((PACK_BLK,), lambda i: (i,)),
        pl.BlockSpec((PACK_BLK,), lambda i: (i,)),
    ],
    out_shape=[
        jax.ShapeDtypeStruct((NTRAIN,), jnp.int32),
        jax.ShapeDtypeStruct((NTRAIN,), jnp.int32),
    ],
)

_mesh = plsc.VectorSubcoreMesh(core_axis_name="c", subcore_axis_name="s")


@functools.partial(
    pl.kernel,
    out_type=jax.ShapeDtypeStruct((NW, L), jnp.float32),
    mesh=_mesh,
    scratch_types=[
        pltpu.VMEM((NTRAIN,), jnp.int32),
        pltpu.VMEM((BPW,), jnp.int32),
        pltpu.VMEM((2, BIT // 2, UCHUNK), jnp.float32),
        pltpu.VMEM((L,), jnp.float32),
        pltpu.SemaphoreType.DMA,
        pltpu.SemaphoreType.DMA,
        pltpu.SemaphoreType.DMA,
    ],
    compiler_params=pltpu.CompilerParams(needs_layout_passes=False),
)
def _mse_bits(
    ut_hbm, ind_hbm, p0_hbm, p1_hbm, out_hbm, pg_v, ind_v, ut_v, acc_v,
    psem, isem, usem,
):
    g = lax.axis_index("c")
    s = lax.axis_index("s")
    wid = s * NC + g
    jbase = pl.multiple_of(g * (BIT // 2), BIT // 2)
    ibase = pl.multiple_of(s * BPW, BPW)

    pcopy0 = pltpu.make_async_copy(p0_hbm, pg_v, psem)
    pcopy1 = pltpu.make_async_copy(p1_hbm, pg_v, psem)

    @pl.when(g == 0)
    def _():
        pcopy0.start()

    @pl.when(g != 0)
    def _():
        pcopy1.start()

    icopy = pltpu.make_async_copy(ind_hbm.at[pl.ds(ibase, BPW)], ind_v, isem)
    icopy.start()

    def u_copy(c, buf):
        return pltpu.make_async_copy(
            ut_hbm.at[
                pl.ds(jbase, BIT // 2),
                pl.ds(pl.multiple_of(ibase + c * UCHUNK, UCHUNK), UCHUNK),
            ],
            ut_v.at[buf],
            usem,
        )

    u_copy(0, 0).start()
    u_copy(1, 1).start()
    icopy.wait()
    pcopy0.wait()

    zero = jnp.zeros((L,), jnp.float32)
    accs = (zero, zero, zero, zero)
    for c in range(NCHUNK):
        buf = c % 2
        u_copy(c, buf).wait()
        cbase = c * UCHUNK

        @plsc.parallel_loop(0, UCHUNK, L, carry=accs)
        def body(k, a, cbase=cbase, buf=buf):
            idx = ind_v[pl.ds(cbase + k, L)]
            p = plsc.load_gather(pg_v, [idx])
            a = list(a)
            for jj in range(BIT // 2):
                neg = jnp.left_shift(p, 31 - jj) < 0
                h = jnp.where(neg, jnp.float32(-1.0), jnp.float32(1.0))
                d = ut_v[buf, jj, pl.ds(k, L)] - h
                a[jj % 4] = a[jj % 4] + d * d
            return tuple(a)

        accs = body
        if c + 2 < NCHUNK:
            u_copy(c + 2, buf).start()
    acc_v[...] = (accs[0] + accs[1]) + (accs[2] + accs[3])
    pltpu.sync_copy(acc_v, out_hbm.at[wid])


def kernel(u, y, ind, H):
    del y
    p0, p1 = _pack(H.T)
    partials = _mse_bits(u.T, ind.astype(jnp.int32), p0, p1)
    return jnp.sum(partials) * (1.0 / (BATCH * BIT))


# speedup vs baseline: 1.1284x; 1.1133x over previous
import functools

import jax
import jax.numpy as jnp
from jax import lax
from jax.experimental import pallas as pl
from jax.experimental.pallas import tpu as pltpu
from jax.experimental.pallas import tpu_sc as plsc

BATCH = 16384
BIT = 64
L = 16
NC = 2
NS = 16
NW = NC * NS
NTRAIN = 100000
PACK_BLK = 12288
PACK_GRID = 9
BPW = BATCH // NS
UCHUNK = 256
NCHUNK = BPW // UCHUNK


def _pack_body(ht_ref, p0_ref, p1_ref):
    x = ht_ref[...]
    j = lax.broadcasted_iota(jnp.int32, (4, BIT), 1)
    q = lax.broadcasted_iota(jnp.int32, (4, BIT), 0)
    inq = (j >= q * 16) & (j < (q + 1) * 16)
    w = jnp.where(inq, jnp.left_shift(jnp.int32(1), j % 16), 0).astype(
        jnp.float32
    )
    dot = lax.dot_general(
        w, x, (((1,), (0,)), ((), ())), preferred_element_type=jnp.float32
    )
    p16 = ((jnp.float32(65535.0) - dot) * jnp.float32(0.5)).astype(jnp.int32)
    p0_ref[...] = p16[0, :] + jnp.left_shift(p16[1, :], 16)
    p1_ref[...] = p16[2, :] + jnp.left_shift(p16[3, :], 16)


_pack = pl.pallas_call(
    _pack_body,
    grid=(PACK_GRID,),
    in_specs=[pl.BlockSpec((BIT, PACK_BLK), lambda i: (0, i))],
    out_specs=[
        pl.BlockSpec((PACK_BLK,), lambda i: (i,)),
        pl.BlockSpec((PACK_BLK,), lambda i: (i,)),
    ],
    out_shape=[
        jax.ShapeDtypeStruct((NTRAIN,), jnp.int32),
        jax.ShapeDtypeStruct((NTRAIN,), jnp.int32),
    ],
)

_mesh = plsc.VectorSubcoreMesh(core_axis_name="c", subcore_axis_name="s")


@functools.partial(
    pl.kernel,
    out_type=jax.ShapeDtypeStruct((NW, L), jnp.float32),
    mesh=_mesh,
    scratch_types=[
        pltpu.VMEM((NTRAIN,), jnp.int32),
        pltpu.VMEM((BPW,), jnp.int32),
        pltpu.VMEM((2, BIT // 2, UCHUNK), jnp.float32),
        pltpu.VMEM((L,), jnp.float32),
        pltpu.VMEM_SHARED((NTRAIN,), jnp.int32),
        pltpu.SemaphoreType.DMA,
        pltpu.SemaphoreType.DMA,
        pltpu.SemaphoreType.DMA,
    ],
    compiler_params=pltpu.CompilerParams(needs_layout_passes=False),
)
def _mse_bits(
    ut_hbm, ind_hbm, p0_hbm, p1_hbm, out_hbm, pg_v, ind_v, ut_v, acc_v,
    p_sh, psem, isem, usem,
):
    g = lax.axis_index("c")
    s = lax.axis_index("s")
    wid = s * NC + g
    jbase = pl.multiple_of(g * (BIT // 2), BIT // 2)
    ibase = pl.multiple_of(s * BPW, BPW)

    pcopy0 = pltpu.make_async_copy(p0_hbm, p_sh, psem)
    pcopy1 = pltpu.make_async_copy(p1_hbm, p_sh, psem)

    @pl.when(jnp.logical_and(s == 0, g == 0))
    def _():
        pcopy0.start()

    @pl.when(jnp.logical_and(s == 0, g != 0))
    def _():
        pcopy1.start()

    icopy = pltpu.make_async_copy(ind_hbm.at[pl.ds(ibase, BPW)], ind_v, isem)
    icopy.start()

    def u_copy(c, buf):
        return pltpu.make_async_copy(
            ut_hbm.at[
                pl.ds(jbase, BIT // 2),
                pl.ds(pl.multiple_of(ibase + c * UCHUNK, UCHUNK), UCHUNK),
            ],
            ut_v.at[buf],
            usem,
        )

    u_copy(0, 0).start()
    u_copy(1, 1).start()

    @pl.when(s == 0)
    def _():
        pcopy0.wait()

    plsc.subcore_barrier()
    pltpu.sync_copy(p_sh, pg_v)
    icopy.wait()

    zero = jnp.zeros((L,), jnp.float32)
    accs = (zero, zero, zero, zero)
    for c in range(NCHUNK):
        buf = c % 2
        u_copy(c, buf).wait()
        cbase = c * UCHUNK

        @plsc.parallel_loop(0, UCHUNK, L, carry=accs)
        def body(k, a, cbase=cbase, buf=buf):
            idx = ind_v[pl.ds(cbase + k, L)]
            p = plsc.load_gather(pg_v, [idx])
            a = list(a)
            for jj in range(BIT // 2):
                neg = jnp.left_shift(p, 31 - jj) < 0
                h = jnp.where(neg, jnp.float32(-1.0), jnp.float32(1.0))
                d = ut_v[buf, jj, pl.ds(k, L)] - h
                a[jj % 4] = a[jj % 4] + d * d
            return tuple(a)

        accs = body
        if c + 2 < NCHUNK:
            u_copy(c + 2, buf).start()
    acc_v[...] = (accs[0] + accs[1]) + (accs[2] + accs[3])
    pltpu.sync_copy(acc_v, out_hbm.at[wid])


def kernel(u, y, ind, H):
    del y
    p0, p1 = _pack(H.T)
    partials = _mse_bits(u.T, ind.astype(jnp.int32), p0, p1)
    return jnp.sum(partials) * (1.0 / (BATCH * BIT))
